# Initial kernel scaffold; baseline (speedup 1.0000x reference)
import jax
import jax.numpy as jnp
from jax.experimental import pallas as pl


def kernel(input, codes, codebook):
    raise NotImplementedError("write your pallas kernel here")


# speedup vs baseline: 1.2276x; 1.2276x over previous
import functools

import jax
import jax.numpy as jnp
from jax import lax
from jax.experimental import pallas as pl
from jax.experimental.pallas import tpu as pltpu
from jax.experimental.pallas import tpu_sc as plsc

NC = 2
NS = 16
NW = NC * NS
L = 16

CHUNK = 128
NBUF = 4
D = 32


def _sc_body(idx_hbm, codes_hbm, cb_hbm, out_hbm,
             idx_v, cb_v, rows_v, outb_v, *sems):
    n_chunks = idx_hbm.shape[1]
    wid = lax.axis_index("s") * NC + lax.axis_index("c")

    pltpu.sync_copy(idx_hbm.at[wid], idx_v)
    pltpu.sync_copy(cb_hbm, cb_v)
    cb = cb_v[...]

    def start_gather(j, b):
        pltpu.async_copy(codes_hbm.at[idx_v.at[j]], rows_v.at[b], sems[b])

    for b in range(NBUF):
        start_gather(b, b)

    def outer(jo):
        for b in range(NBUF):
            j = jo + b
            pltpu.make_async_copy(
                codes_hbm.at[idx_v.at[j]], rows_v.at[b], sems[b]
            ).wait()

            @pl.loop(0, CHUNK, unroll=4)
            def decode_row(r):
                for h in range(D // L):
                    codes16 = rows_v[b, r, pl.ds(h * L, L)]
                    dec = jnp.take_along_axis(
                        cb, codes16, axis=0,
                        mode=lax.GatherScatterMode.PROMISE_IN_BOUNDS)
                    outb_v[r, pl.ds(h * L, L)] = dec

            pltpu.sync_copy(outb_v, out_hbm.at[wid, j])

            @pl.when(j + NBUF < n_chunks)
            def _():
                start_gather(j + NBUF, b)

    pl.loop(0, n_chunks, step=NBUF)(outer)


@jax.jit
def _sc_decode(idx, codes, codebook):
    n_chunks = idx.shape[1]
    mesh = plsc.VectorSubcoreMesh(core_axis_name="c", subcore_axis_name="s")
    return pl.kernel(
        _sc_body,
        out_type=jax.ShapeDtypeStruct((NW, n_chunks, CHUNK, D), jnp.float32),
        mesh=mesh,
        scratch_types=[
            pltpu.VMEM((n_chunks, CHUNK), jnp.int32),
            pltpu.VMEM((L,), jnp.float32),
            pltpu.VMEM((NBUF, CHUNK, D), jnp.int32),
            pltpu.VMEM((CHUNK, D), jnp.float32),
        ] + [pltpu.SemaphoreType.DMA] * NBUF,
        compiler_params=pltpu.CompilerParams(use_tc_tiling_on_sc=False),
    )(idx, codes, codebook)


def kernel(input, codes, codebook):
    b, h = input.shape
    rows = b * h
    idx = input.reshape(NW, rows // NW // CHUNK, CHUNK)
    out = _sc_decode(idx, codes, codebook)
    return out.reshape(b, h, codes.shape[1])


# speedup vs baseline: 1.6717x; 1.3617x over previous
import functools

import jax
import jax.numpy as jnp
from jax import lax
from jax.experimental import pallas as pl
from jax.experimental.pallas import tpu as pltpu
from jax.experimental.pallas import tpu_sc as plsc

NC = 2
NS = 16
NW = NC * NS
L = 16

CHUNK = 128
NBUF = 4
D = 32
GPC = CHUNK * D // L


def _sc_body(idx_hbm, codes_hbm, cb_hbm, out_hbm,
             idx_v, cb_v, rows_v, outb_v, *sems):
    n_chunks = idx_hbm.shape[1]
    gsems = sems[:NBUF]
    osems = sems[NBUF:]
    wid = lax.axis_index("s") * NC + lax.axis_index("c")
    oc = CHUNK * D // 128

    pltpu.sync_copy(idx_hbm.at[wid], idx_v)
    pltpu.sync_copy(cb_hbm, cb_v)
    cb = cb_v[...]

    def start_gather(j, b):
        pltpu.async_copy(codes_hbm.at[idx_v.at[j]], rows_v.at[b], gsems[b])

    for b in range(NBUF):
        start_gather(b, b)

    def outer(jo):
        for b in range(NBUF):
            j = jo + b
            pltpu.make_async_copy(
                codes_hbm.at[idx_v.at[j]], rows_v.at[b], gsems[b]
            ).wait()

            @pl.when(j >= NBUF)
            def _():
                pltpu.make_async_copy(
                    outb_v.at[b], out_hbm.at[wid, pl.ds(0, oc)], osems[b]
                ).wait()

            @pl.loop(0, GPC, unroll=8)
            def decode(g):
                codes16 = rows_v[b, g // 2, pl.ds((g % 2) * L, L)]
                dec = jnp.take_along_axis(
                    cb, codes16, axis=0,
                    mode=lax.GatherScatterMode.PROMISE_IN_BOUNDS)
                outb_v[b, g // 8, pl.ds((g % 8) * L, L)] = dec

            pltpu.async_copy(
                outb_v.at[b], out_hbm.at[wid, pl.ds(j * oc, oc)], osems[b])

            @pl.when(j + NBUF < n_chunks)
            def _():
                start_gather(j + NBUF, b)

    pl.loop(0, n_chunks, step=NBUF)(outer)

    for b in range(NBUF):
        pltpu.make_async_copy(
            outb_v.at[b], out_hbm.at[wid, pl.ds(0, oc)], osems[b]
        ).wait()


@jax.jit
def _sc_decode(idx, codes, codebook):
    n_chunks = idx.shape[1]
    oc = CHUNK * D // 128
    mesh = plsc.VectorSubcoreMesh(core_axis_name="c", subcore_axis_name="s")
    return pl.kernel(
        _sc_body,
        out_type=jax.ShapeDtypeStruct((NW, n_chunks * oc, 128), jnp.float32),
        mesh=mesh,
        scratch_types=[
            pltpu.VMEM((n_chunks, CHUNK), jnp.int32),
            pltpu.VMEM((L,), jnp.float32),
            pltpu.VMEM((NBUF, CHUNK, D), jnp.int32),
            pltpu.VMEM((NBUF, oc, 128), jnp.float32),
        ] + [pltpu.SemaphoreType.DMA] * (2 * NBUF),
        compiler_params=pltpu.CompilerParams(use_tc_tiling_on_sc=False),
    )(idx, codes, codebook)


def kernel(input, codes, codebook):
    b, h = input.shape
    rows = b * h
    idx = input.reshape(NW, rows // NW // CHUNK, CHUNK)
    out = _sc_decode(idx, codes, codebook)
    return out.reshape(b, h, codes.shape[1])
